# baseline (speedup 1.0000x reference)
import functools

import jax
import jax.numpy as jnp
from jax import lax
from jax.experimental import pallas as pl
from jax.experimental.pallas import tpu as pltpu
from jax.experimental.pallas import tpu_sc as plsc

_N = 10000
_E = 320000
_F = 128
_NC = 2
_NS = 16
_NW = _NC * _NS
_EPT = _E // _NW
_CH = 80
_NCHK = _EPT // _CH
_NHEX = (_NCHK - 5) // 6
_NP = 10240
_RPT = _NP // _NS

_R = 5000
_G = _N // _R

_mesh = plsc.VectorSubcoreMesh(core_axis_name="c", subcore_axis_name="s")
_scparams = pltpu.CompilerParams(use_tc_tiling_on_sc=False,
                                 needs_layout_passes=False)


@functools.partial(
    pl.kernel,
    mesh=_mesh,
    out_type=[
        jax.ShapeDtypeStruct((_NC, _NP, _F), jnp.float32),
        jax.ShapeDtypeStruct((_NC, _NP, 16), jnp.float32),
    ],
    scratch_types=[
        pltpu.VMEM_SHARED((_NP, _F), jnp.float32),
        pltpu.VMEM_SHARED((_NP, 16), jnp.float32),
        pltpu.VMEM((2, _CH), jnp.int32),
        pltpu.VMEM((2, _CH), jnp.int32),
        pltpu.VMEM((2, _CH), jnp.int32),
        pltpu.VMEM((2, _CH), jnp.int32),
        pltpu.VMEM((2, _CH), jnp.int32),
        pltpu.VMEM((2, _CH), jnp.int32),
        pltpu.VMEM((3, _CH, _F), jnp.float32),
        pltpu.VMEM((_CH, 16), jnp.float32),
        pltpu.SemaphoreType.DMA,
        pltpu.SemaphoreType.DMA,
        pltpu.SemaphoreType.DMA,
        pltpu.SemaphoreType.DMA,
        pltpu.SemaphoreType.DMA,
        pltpu.SemaphoreType.DMA,
        pltpu.SemaphoreType.DMA,
        pltpu.SemaphoreType.DMA,
        pltpu.SemaphoreType.DMA,
        pltpu.SemaphoreType.DMA,
        pltpu.SemaphoreType.DMA,
        pltpu.SemaphoreType.DMA,
    ],
    compiler_params=_scparams,
)
def _sc_segsum_counts(src_hbm, dst_hbm, x_hbm, zrow_hbm, zc16_hbm,
                      p_out, cnt_out,
                      acc_sh, cnt_sh, sa, da, sb, db, sc_, dc, rows, ones16,
                      g0, g1, g2, ss0, ss1, ss2, so0, so1, so2,
                      semia, semib, semic):
    c = lax.axis_index("c")
    s = lax.axis_index("s")
    wid = c * _NS + s
    r0 = s * _RPT

    pltpu.sync_copy(src_hbm.at[wid, pl.ds(0, 2)], sa)
    pltpu.sync_copy(dst_hbm.at[wid, pl.ds(0, 2)], da)
    pltpu.async_copy(src_hbm.at[wid, pl.ds(2, 2)], sb, semib)
    pltpu.async_copy(dst_hbm.at[wid, pl.ds(2, 2)], db, semib)
    pltpu.async_copy(src_hbm.at[wid, pl.ds(4, 2)], sc_, semic)
    pltpu.async_copy(dst_hbm.at[wid, pl.ds(4, 2)], dc, semic)
    pltpu.async_copy(x_hbm.at[sa.at[0]], rows.at[0], g0)
    pltpu.async_copy(x_hbm.at[sa.at[1]], rows.at[1], g1)

    o16 = jnp.ones((16,), jnp.float32)

    def _ones_row(r, carry):
        ones16[r, :] = o16
        return carry

    lax.fori_loop(0, _CH, _ones_row, 0)

    pltpu.sync_copy(zrow_hbm, acc_sh.at[pl.ds(r0, _RPT)])
    pltpu.sync_copy(zc16_hbm, cnt_sh.at[pl.ds(r0, _RPT)])
    plsc.subcore_barrier()

    def _hex(i, carry):
        j = 6 * i
        pltpu.make_async_copy(x_hbm.at[sa.at[0]], rows.at[0], g0).wait()
        s0 = pltpu.async_copy(rows.at[0], acc_sh.at[da.at[0]], ss0, add=True)
        o0 = pltpu.async_copy(ones16, cnt_sh.at[da.at[0]], so0, add=True)
        pltpu.make_async_copy(src_hbm.at[wid, pl.ds(0, 2)], sb, semib).wait()
        pltpu.make_async_copy(dst_hbm.at[wid, pl.ds(0, 2)], db, semib).wait()
        pltpu.async_copy(x_hbm.at[sb.at[0]], rows.at[2], g2)
        pltpu.make_async_copy(x_hbm.at[sa.at[1]], rows.at[1], g1).wait()
        s1 = pltpu.async_copy(rows.at[1], acc_sh.at[da.at[1]], ss1, add=True)
        o1 = pltpu.async_copy(ones16, cnt_sh.at[da.at[1]], so1, add=True)
        s0.wait()
        o0.wait()
        pltpu.async_copy(x_hbm.at[sb.at[1]], rows.at[0], g0)
        pltpu.make_async_copy(x_hbm.at[sb.at[0]], rows.at[2], g2).wait()
        s2 = pltpu.async_copy(rows.at[2], acc_sh.at[db.at[0]], ss2, add=True)
        o2 = pltpu.async_copy(ones16, cnt_sh.at[db.at[0]], so2, add=True)
        s1.wait()
        o1.wait()
        pltpu.make_async_copy(src_hbm.at[wid, pl.ds(0, 2)], sc_, semic).wait()
        pltpu.make_async_copy(dst_hbm.at[wid, pl.ds(0, 2)], dc, semic).wait()
        pltpu.async_copy(x_hbm.at[sc_.at[0]], rows.at[1], g1)
        pltpu.async_copy(src_hbm.at[wid, pl.ds(j + 6, 2)], sa, semia)
        pltpu.async_copy(dst_hbm.at[wid, pl.ds(j + 6, 2)], da, semia)
        pltpu.make_async_copy(x_hbm.at[sb.at[1]], rows.at[0], g0).wait()
        s3 = pltpu.async_copy(rows.at[0], acc_sh.at[db.at[1]], ss0, add=True)
        o3 = pltpu.async_copy(ones16, cnt_sh.at[db.at[1]], so0, add=True)
        s2.wait()
        o2.wait()
        pltpu.async_copy(x_hbm.at[sc_.at[1]], rows.at[2], g2)
        pltpu.make_async_copy(x_hbm.at[sc_.at[0]], rows.at[1], g1).wait()
        s4 = pltpu.async_copy(rows.at[1], acc_sh.at[dc.at[0]], ss1, add=True)
        o4 = pltpu.async_copy(ones16, cnt_sh.at[dc.at[0]], so1, add=True)
        s3.wait()
        o3.wait()
        pltpu.make_async_copy(src_hbm.at[wid, pl.ds(0, 2)], sa, semia).wait()
        pltpu.make_async_copy(dst_hbm.at[wid, pl.ds(0, 2)], da, semia).wait()
        pltpu.async_copy(x_hbm.at[sa.at[0]], rows.at[0], g0)
        pltpu.async_copy(src_hbm.at[wid, pl.ds(j + 8, 2)], sb, semib)
        pltpu.async_copy(dst_hbm.at[wid, pl.ds(j + 8, 2)], db, semib)
        pltpu.make_async_copy(x_hbm.at[sc_.at[1]], rows.at[2], g2).wait()
        s5 = pltpu.async_copy(rows.at[2], acc_sh.at[dc.at[1]], ss2, add=True)
        o5 = pltpu.async_copy(ones16, cnt_sh.at[dc.at[1]], so2, add=True)
        s4.wait()
        o4.wait()
        pltpu.async_copy(x_hbm.at[sa.at[1]], rows.at[1], g1)

        @pl.when(i < _NHEX - 1)
        def _():
            pltpu.async_copy(src_hbm.at[wid, pl.ds(j + 10, 2)], sc_, semic)
            pltpu.async_copy(dst_hbm.at[wid, pl.ds(j + 10, 2)], dc, semic)

        s5.wait()
        o5.wait()
        return carry

    lax.fori_loop(0, _NHEX, _hex, 0)
    jt = 6 * _NHEX
    pltpu.make_async_copy(x_hbm.at[sa.at[0]], rows.at[0], g0).wait()
    s0 = pltpu.async_copy(rows.at[0], acc_sh.at[da.at[0]], ss0, add=True)
    o0 = pltpu.async_copy(ones16, cnt_sh.at[da.at[0]], so0, add=True)
    pltpu.make_async_copy(src_hbm.at[wid, pl.ds(0, 2)], sb, semib).wait()
    pltpu.make_async_copy(dst_hbm.at[wid, pl.ds(0, 2)], db, semib).wait()
    pltpu.async_copy(x_hbm.at[sb.at[0]], rows.at[2], g2)
    pltpu.make_async_copy(x_hbm.at[sa.at[1]], rows.at[1], g1).wait()
    s1 = pltpu.async_copy(rows.at[1], acc_sh.at[da.at[1]], ss1, add=True)
    o1 = pltpu.async_copy(ones16, cnt_sh.at[da.at[1]], so1, add=True)
    s0.wait()
    o0.wait()
    pltpu.async_copy(x_hbm.at[sb.at[1]], rows.at[0], g0)
    pltpu.make_async_copy(x_hbm.at[sb.at[0]], rows.at[2], g2).wait()
    s2 = pltpu.async_copy(rows.at[2], acc_sh.at[db.at[0]], ss2, add=True)
    o2 = pltpu.async_copy(ones16, cnt_sh.at[db.at[0]], so2, add=True)
    s1.wait()
    o1.wait()
    pltpu.sync_copy(src_hbm.at[wid, pl.ds(_NCHK - 1, 1)], sa.at[pl.ds(0, 1)])
    pltpu.sync_copy(dst_hbm.at[wid, pl.ds(_NCHK - 1, 1)], da.at[pl.ds(0, 1)])
    pltpu.async_copy(x_hbm.at[sa.at[0]], rows.at[1], g1)
    pltpu.make_async_copy(x_hbm.at[sb.at[1]], rows.at[0], g0).wait()
    s3 = pltpu.async_copy(rows.at[0], acc_sh.at[db.at[1]], ss0, add=True)
    o3 = pltpu.async_copy(ones16, cnt_sh.at[db.at[1]], so0, add=True)
    s2.wait()
    o2.wait()
    pltpu.make_async_copy(x_hbm.at[sa.at[0]], rows.at[1], g1).wait()
    s4 = pltpu.async_copy(rows.at[1], acc_sh.at[da.at[0]], ss1, add=True)
    o4 = pltpu.async_copy(ones16, cnt_sh.at[da.at[0]], so1, add=True)
    s3.wait()
    o3.wait()
    s4.wait()
    o4.wait()
    plsc.subcore_barrier()

    pltpu.sync_copy(acc_sh.at[pl.ds(r0, _RPT)], p_out.at[c, pl.ds(r0, _RPT)])
    pltpu.sync_copy(cnt_sh.at[pl.ds(r0, _RPT)], cnt_out.at[c, pl.ds(r0, _RPT)])


@functools.partial(
    pl.kernel,
    mesh=_mesh,
    out_type=[
        jax.ShapeDtypeStruct((_NC, _NP, _F), jnp.float32),
        jax.ShapeDtypeStruct((_NC, _NP), jnp.float32),
    ],
    scratch_types=[
        pltpu.VMEM_SHARED((_NP, _F), jnp.float32),
        pltpu.VMEM_SHARED((_NP,), jnp.float32),
        pltpu.VMEM_SHARED((_NP,), jnp.float32),
        pltpu.VMEM((2, _CH), jnp.int32),
        pltpu.VMEM((2, _CH), jnp.int32),
        pltpu.VMEM((2, _CH), jnp.int32),
        pltpu.VMEM((2, _CH), jnp.int32),
        pltpu.VMEM((2, _CH), jnp.int32),
        pltpu.VMEM((2, _CH), jnp.int32),
        pltpu.VMEM((3, _CH, _F), jnp.float32),
        pltpu.VMEM((3, _CH), jnp.float32),
        pltpu.VMEM((_RPT,), jnp.float32),
        pltpu.VMEM((80, 16), jnp.float32),
        pltpu.VMEM((80, 16), jnp.float32),
        pltpu.VMEM((_NP,), jnp.float32),
        pltpu.SemaphoreType.DMA,
        pltpu.SemaphoreType.DMA,
        pltpu.SemaphoreType.DMA,
        pltpu.SemaphoreType.DMA,
        pltpu.SemaphoreType.DMA,
        pltpu.SemaphoreType.DMA,
        pltpu.SemaphoreType.DMA,
        pltpu.SemaphoreType.DMA,
        pltpu.SemaphoreType.DMA,
        pltpu.SemaphoreType.DMA,
        pltpu.SemaphoreType.DMA,
        pltpu.SemaphoreType.DMA,
    ],
    compiler_params=_scparams,
)
def _sc_segsum_cvec(src_hbm, dst_hbm, h_hbm, cnt_hbm, zrow_hbm,
                    q_out, cvec_out,
                    acc_sh, cvec_sh, inv_sh, sa, da, sb, db, sc_, dc, rows,
                    w, tmp1, cbuf0, cbuf1, invloc,
                    g0, g1, g2, ss0, ss1, ss2, sw0, sw1, sw2,
                    semia, semib, semic):
    c = lax.axis_index("c")
    s = lax.axis_index("s")
    wid = c * _NS + s
    r0 = s * _RPT

    pltpu.sync_copy(src_hbm.at[wid, pl.ds(0, 2)], sa)
    pltpu.sync_copy(dst_hbm.at[wid, pl.ds(0, 2)], da)
    pltpu.async_copy(src_hbm.at[wid, pl.ds(2, 2)], sb, semib)
    pltpu.async_copy(dst_hbm.at[wid, pl.ds(2, 2)], db, semib)
    pltpu.async_copy(src_hbm.at[wid, pl.ds(4, 2)], sc_, semic)
    pltpu.async_copy(dst_hbm.at[wid, pl.ds(4, 2)], dc, semic)
    pltpu.async_copy(h_hbm.at[sa.at[0]], rows.at[0], g0)
    pltpu.async_copy(h_hbm.at[sa.at[1]], rows.at[1], g1)

    zidx = jnp.zeros((16,), jnp.int32)
    i16 = lax.iota(jnp.int32, 16)
    for q in range(8):
        pltpu.sync_copy(cnt_hbm.at[0, pl.ds(r0 + q * 80, 80)], cbuf0)
        pltpu.sync_copy(cnt_hbm.at[1, pl.ds(r0 + q * 80, 80)], cbuf1)

        def _inv_grp(g, carry):
            ridx = i16 + g * 16
            c0 = plsc.load_gather(cbuf0, [ridx, zidx])
            c1 = plsc.load_gather(cbuf1, [ridx, zidx])
            tmp1[pl.ds(q * 80 + g * 16, 16)] = 1.0 / jnp.maximum(c0 + c1, 1.0)
            return carry

        lax.fori_loop(0, 5, _inv_grp, 0)
    pltpu.sync_copy(tmp1, inv_sh.at[pl.ds(r0, _RPT)])

    pltpu.sync_copy(zrow_hbm, acc_sh.at[pl.ds(r0, _RPT)])

    def _z1(i, carry):
        tmp1[pl.ds(i * 16, 16)] = jnp.zeros((16,), jnp.float32)
        return carry

    lax.fori_loop(0, _RPT // 16, _z1, 0)
    pltpu.sync_copy(tmp1, cvec_sh.at[pl.ds(r0, _RPT)])
    plsc.subcore_barrier()
    pltpu.sync_copy(inv_sh, invloc)

    def _wvals(b, dref, ch):
        for k in range(_CH // 16):
            d16 = dref[ch, pl.ds(k * 16, 16)]
            w[b, pl.ds(k * 16, 16)] = plsc.load_gather(invloc, [d16])

    def _chunk(b, sref, dref, ch, ssem, wsem):
        pltpu.make_async_copy(h_hbm.at[sref.at[ch]], rows.at[b], [g0, g1, g2][b]).wait()
        sD = pltpu.async_copy(rows.at[b], acc_sh.at[dref.at[ch]], ssem,
                              add=True)
        _wvals(b, dref, ch)
        wD = pltpu.async_copy(w.at[b], cvec_sh.at[sref.at[ch]], wsem,
                              add=True)
        return sD, wD

    def _hex(i, carry):
        j = 6 * i
        s0, w0 = _chunk(0, sa, da, 0, ss0, sw0)
        pltpu.make_async_copy(src_hbm.at[wid, pl.ds(0, 2)], sb, semib).wait()
        pltpu.make_async_copy(dst_hbm.at[wid, pl.ds(0, 2)], db, semib).wait()
        pltpu.async_copy(h_hbm.at[sb.at[0]], rows.at[2], g2)
        s1, w1 = _chunk(1, sa, da, 1, ss1, sw1)
        s0.wait()
        w0.wait()
        pltpu.async_copy(h_hbm.at[sb.at[1]], rows.at[0], g0)
        s2, w2 = _chunk(2, sb, db, 0, ss2, sw2)
        s1.wait()
        w1.wait()
        pltpu.make_async_copy(src_hbm.at[wid, pl.ds(0, 2)], sc_, semic).wait()
        pltpu.make_async_copy(dst_hbm.at[wid, pl.ds(0, 2)], dc, semic).wait()
        pltpu.async_copy(h_hbm.at[sc_.at[0]], rows.at[1], g1)
        pltpu.async_copy(src_hbm.at[wid, pl.ds(j + 6, 2)], sa, semia)
        pltpu.async_copy(dst_hbm.at[wid, pl.ds(j + 6, 2)], da, semia)
        s3, w3 = _chunk(0, sb, db, 1, ss0, sw0)
        s2.wait()
        w2.wait()
        pltpu.async_copy(h_hbm.at[sc_.at[1]], rows.at[2], g2)
        s4, w4 = _chunk(1, sc_, dc, 0, ss1, sw1)
        s3.wait()
        w3.wait()
        pltpu.make_async_copy(src_hbm.at[wid, pl.ds(0, 2)], sa, semia).wait()
        pltpu.make_async_copy(dst_hbm.at[wid, pl.ds(0, 2)], da, semia).wait()
        pltpu.async_copy(h_hbm.at[sa.at[0]], rows.at[0], g0)
        pltpu.async_copy(src_hbm.at[wid, pl.ds(j + 8, 2)], sb, semib)
        pltpu.async_copy(dst_hbm.at[wid, pl.ds(j + 8, 2)], db, semib)
        s5, w5 = _chunk(2, sc_, dc, 1, ss2, sw2)
        s4.wait()
        w4.wait()
        pltpu.async_copy(h_hbm.at[sa.at[1]], rows.at[1], g1)

        @pl.when(i < _NHEX - 1)
        def _():
            pltpu.async_copy(src_hbm.at[wid, pl.ds(j + 10, 2)], sc_, semic)
            pltpu.async_copy(dst_hbm.at[wid, pl.ds(j + 10, 2)], dc, semic)

        s5.wait()
        w5.wait()
        return carry

    lax.fori_loop(0, _NHEX, _hex, 0)
    s0, w0 = _chunk(0, sa, da, 0, ss0, sw0)
    pltpu.make_async_copy(src_hbm.at[wid, pl.ds(0, 2)], sb, semib).wait()
    pltpu.make_async_copy(dst_hbm.at[wid, pl.ds(0, 2)], db, semib).wait()
    pltpu.async_copy(h_hbm.at[sb.at[0]], rows.at[2], g2)
    s1, w1 = _chunk(1, sa, da, 1, ss1, sw1)
    s0.wait()
    w0.wait()
    pltpu.async_copy(h_hbm.at[sb.at[1]], rows.at[0], g0)
    s2, w2 = _chunk(2, sb, db, 0, ss2, sw2)
    s1.wait()
    w1.wait()
    pltpu.sync_copy(src_hbm.at[wid, pl.ds(_NCHK - 1, 1)], sa.at[pl.ds(0, 1)])
    pltpu.sync_copy(dst_hbm.at[wid, pl.ds(_NCHK - 1, 1)], da.at[pl.ds(0, 1)])
    pltpu.async_copy(h_hbm.at[sa.at[0]], rows.at[1], g1)
    s3, w3 = _chunk(0, sb, db, 1, ss0, sw0)
    s2.wait()
    w2.wait()
    s4, w4 = _chunk(1, sa, da, 0, ss1, sw1)
    s3.wait()
    w3.wait()
    s4.wait()
    w4.wait()
    plsc.subcore_barrier()

    pltpu.sync_copy(acc_sh.at[pl.ds(r0, _RPT)], q_out.at[c, pl.ds(r0, _RPT)])
    pltpu.sync_copy(cvec_sh.at[pl.ds(r0, _RPT)], cvec_out.at[c, pl.ds(r0, _RPT)])


def _dotT(a, b):
    return lax.dot_general(a, b, (((1,), (1,)), ((), ())),
                           preferred_element_type=jnp.float32)


def _dense1_body(p_ref, cnt_ref, x_ref, wl_ref, bl_ref, wr_ref, h_ref):
    cnt = cnt_ref[0][:, 0:1] + cnt_ref[1][:, 0:1]
    inv = 1.0 / jnp.maximum(cnt, 1.0)
    agg = (p_ref[0] + p_ref[1]) * inv
    h = _dotT(agg, wl_ref[...]) + _dotT(x_ref[...], wr_ref[...]) + bl_ref[...]
    h_ref[...] = jnp.maximum(h, 0.0)


def _dense2_body(q_ref, cv_ref, h1_ref, cnt_ref, wl2_ref, bl2_ref, wr2_ref,
                 wl3_ref, bl3_ref, wr3_ref, wc_ref, bc_ref,
                 out_ref, g1_acc, g2_acc):
    i = pl.program_id(0)
    cnt = cnt_ref[0][:, 0:1] + cnt_ref[1][:, 0:1]
    inv = 1.0 / jnp.maximum(cnt, 1.0)
    agg = (q_ref[0] + q_ref[1]) * inv
    h2 = _dotT(agg, wl2_ref[...]) + _dotT(h1_ref[...], wr2_ref[...]) + bl2_ref[...]
    h2 = jnp.maximum(h2, 0.0)
    cv = cv_ref[:, 0:1] + cv_ref[:, 1:2]
    part1 = jnp.sum(cv * h2, axis=0, keepdims=True)
    part2 = jnp.sum(h2, axis=0, keepdims=True)

    @pl.when(i == 0)
    def _():
        g1_acc[...] = jnp.zeros_like(g1_acc)
        g2_acc[...] = jnp.zeros_like(g2_acc)

    g1_acc[...] += part1
    g2_acc[...] += part2

    @pl.when(i == _G - 1)
    def _():
        g1 = g1_acc[...] * (1.0 / _N)
        g2 = g2_acc[...] * (1.0 / _N)
        gm = _dotT(g1, wl3_ref[...]) + bl3_ref[...] + _dotT(g2, wr3_ref[...])
        out_ref[...] = _dotT(gm, wc_ref[...]) + bc_ref[...]


def kernel(x, edge_index, Wl1, bl1, Wr1, Wl2, bl2, Wr2, Wl3, bl3, Wr3, Wc, bc):
    src_r = edge_index[0].reshape(_NW, _NCHK, _CH)
    dst_r = edge_index[1].reshape(_NW, _NCHK, _CH)
    zrow = jnp.zeros((_RPT, _F), jnp.float32)
    zc16 = jnp.zeros((_RPT, 16), jnp.float32)

    p1, cnt16 = _sc_segsum_counts(src_r, dst_r, x, zrow, zc16)

    wfull = pl.BlockSpec((_F, _F), lambda i: (0, 0))
    bfull = pl.BlockSpec((1, _F), lambda i: (0, 0))
    rowblk = pl.BlockSpec((_R, _F), lambda i: (i, 0))
    pblk = pl.BlockSpec((_NC, _R, _F), lambda i: (0, i, 0))
    cblk = pl.BlockSpec((_NC, _R, 16), lambda i: (0, i, 0))

    h1 = pl.pallas_call(
        _dense1_body,
        grid=(_G,),
        in_specs=[pblk, cblk, rowblk, wfull, bfull, wfull],
        out_specs=rowblk,
        out_shape=jax.ShapeDtypeStruct((_N, _F), jnp.float32),
    )(p1, cnt16, x, Wl1, bl1.reshape(1, _F), Wr1)

    q2, cvec = _sc_segsum_cvec(src_r, dst_r, h1, cnt16, zrow)

    cvblk = pl.BlockSpec((_R, _NC), lambda i: (i, 0))
    out = pl.pallas_call(
        _dense2_body,
        grid=(_G,),
        in_specs=[pblk, cvblk, rowblk, cblk, wfull, bfull, wfull,
                  wfull, bfull, wfull,
                  pl.BlockSpec((Wc.shape[0], _F), lambda i: (0, 0)),
                  pl.BlockSpec((1, Wc.shape[0]), lambda i: (0, 0))],
        out_specs=pl.BlockSpec((1, Wc.shape[0]), lambda i: (0, 0)),
        out_shape=jax.ShapeDtypeStruct((1, Wc.shape[0]), jnp.float32),
        scratch_shapes=[pltpu.VMEM((1, _F), jnp.float32),
                        pltpu.VMEM((1, _F), jnp.float32)],
    )(q2, cvec.T, h1, cnt16, Wl2, bl2.reshape(1, _F), Wr2,
      Wl3, bl3.reshape(1, _F), Wr3, Wc, bc.reshape(1, -1))

    return out


# speedup vs baseline: 1.0046x; 1.0046x over previous
import functools

import jax
import jax.numpy as jnp
from jax import lax
from jax.experimental import pallas as pl
from jax.experimental.pallas import tpu as pltpu
from jax.experimental.pallas import tpu_sc as plsc

_N = 10000
_E = 320000
_F = 128
_NC = 2
_NS = 16
_NW = _NC * _NS
_EPT = _E // _NW
_CH = 80
_NCHK = _EPT // _CH
_NHEX = (_NCHK - 5) // 6
_NP = 10240
_RPT = _NP // _NS

_R = 2000
_G = _N // _R

_mesh = plsc.VectorSubcoreMesh(core_axis_name="c", subcore_axis_name="s")
_scparams = pltpu.CompilerParams(use_tc_tiling_on_sc=False,
                                 needs_layout_passes=False)


@functools.partial(
    pl.kernel,
    mesh=_mesh,
    out_type=[
        jax.ShapeDtypeStruct((_NC, _NP, _F), jnp.float32),
        jax.ShapeDtypeStruct((_NC, _NP, 16), jnp.float32),
    ],
    scratch_types=[
        pltpu.VMEM_SHARED((_NP, _F), jnp.float32),
        pltpu.VMEM_SHARED((_NP, 16), jnp.float32),
        pltpu.VMEM((2, _CH), jnp.int32),
        pltpu.VMEM((2, _CH), jnp.int32),
        pltpu.VMEM((2, _CH), jnp.int32),
        pltpu.VMEM((2, _CH), jnp.int32),
        pltpu.VMEM((2, _CH), jnp.int32),
        pltpu.VMEM((2, _CH), jnp.int32),
        pltpu.VMEM((3, _CH, _F), jnp.float32),
        pltpu.VMEM((_CH, 16), jnp.float32),
        pltpu.SemaphoreType.DMA,
        pltpu.SemaphoreType.DMA,
        pltpu.SemaphoreType.DMA,
        pltpu.SemaphoreType.DMA,
        pltpu.SemaphoreType.DMA,
        pltpu.SemaphoreType.DMA,
        pltpu.SemaphoreType.DMA,
        pltpu.SemaphoreType.DMA,
        pltpu.SemaphoreType.DMA,
        pltpu.SemaphoreType.DMA,
        pltpu.SemaphoreType.DMA,
        pltpu.SemaphoreType.DMA,
    ],
    compiler_params=_scparams,
)
def _sc_segsum_counts(src_hbm, dst_hbm, x_hbm, zrow_hbm, zc16_hbm,
                      p_out, cnt_out,
                      acc_sh, cnt_sh, sa, da, sb, db, sc_, dc, rows, ones16,
                      g0, g1, g2, ss0, ss1, ss2, so0, so1, so2,
                      semia, semib, semic):
    c = lax.axis_index("c")
    s = lax.axis_index("s")
    wid = c * _NS + s
    r0 = s * _RPT

    pltpu.sync_copy(src_hbm.at[wid, pl.ds(0, 2)], sa)
    pltpu.sync_copy(dst_hbm.at[wid, pl.ds(0, 2)], da)
    pltpu.async_copy(src_hbm.at[wid, pl.ds(2, 2)], sb, semib)
    pltpu.async_copy(dst_hbm.at[wid, pl.ds(2, 2)], db, semib)
    pltpu.async_copy(src_hbm.at[wid, pl.ds(4, 2)], sc_, semic)
    pltpu.async_copy(dst_hbm.at[wid, pl.ds(4, 2)], dc, semic)
    pltpu.async_copy(x_hbm.at[sa.at[0]], rows.at[0], g0)
    pltpu.async_copy(x_hbm.at[sa.at[1]], rows.at[1], g1)

    o16 = jnp.ones((16,), jnp.float32)

    def _ones_row(r, carry):
        ones16[r, :] = o16
        return carry

    lax.fori_loop(0, _CH, _ones_row, 0)

    pltpu.sync_copy(zrow_hbm, acc_sh.at[pl.ds(r0, _RPT)])
    pltpu.sync_copy(zc16_hbm, cnt_sh.at[pl.ds(r0, _RPT)])
    plsc.subcore_barrier()

    def _hex(i, carry):
        j = 6 * i
        pltpu.make_async_copy(x_hbm.at[sa.at[0]], rows.at[0], g0).wait()
        s0 = pltpu.async_copy(rows.at[0], acc_sh.at[da.at[0]], ss0, add=True)
        o0 = pltpu.async_copy(ones16, cnt_sh.at[da.at[0]], so0, add=True)
        pltpu.make_async_copy(src_hbm.at[wid, pl.ds(0, 2)], sb, semib).wait()
        pltpu.make_async_copy(dst_hbm.at[wid, pl.ds(0, 2)], db, semib).wait()
        pltpu.async_copy(x_hbm.at[sb.at[0]], rows.at[2], g2)
        pltpu.make_async_copy(x_hbm.at[sa.at[1]], rows.at[1], g1).wait()
        s1 = pltpu.async_copy(rows.at[1], acc_sh.at[da.at[1]], ss1, add=True)
        o1 = pltpu.async_copy(ones16, cnt_sh.at[da.at[1]], so1, add=True)
        s0.wait()
        o0.wait()
        pltpu.async_copy(x_hbm.at[sb.at[1]], rows.at[0], g0)
        pltpu.make_async_copy(x_hbm.at[sb.at[0]], rows.at[2], g2).wait()
        s2 = pltpu.async_copy(rows.at[2], acc_sh.at[db.at[0]], ss2, add=True)
        o2 = pltpu.async_copy(ones16, cnt_sh.at[db.at[0]], so2, add=True)
        s1.wait()
        o1.wait()
        pltpu.make_async_copy(src_hbm.at[wid, pl.ds(0, 2)], sc_, semic).wait()
        pltpu.make_async_copy(dst_hbm.at[wid, pl.ds(0, 2)], dc, semic).wait()
        pltpu.async_copy(x_hbm.at[sc_.at[0]], rows.at[1], g1)
        pltpu.async_copy(src_hbm.at[wid, pl.ds(j + 6, 2)], sa, semia)
        pltpu.async_copy(dst_hbm.at[wid, pl.ds(j + 6, 2)], da, semia)
        pltpu.make_async_copy(x_hbm.at[sb.at[1]], rows.at[0], g0).wait()
        s3 = pltpu.async_copy(rows.at[0], acc_sh.at[db.at[1]], ss0, add=True)
        o3 = pltpu.async_copy(ones16, cnt_sh.at[db.at[1]], so0, add=True)
        s2.wait()
        o2.wait()
        pltpu.async_copy(x_hbm.at[sc_.at[1]], rows.at[2], g2)
        pltpu.make_async_copy(x_hbm.at[sc_.at[0]], rows.at[1], g1).wait()
        s4 = pltpu.async_copy(rows.at[1], acc_sh.at[dc.at[0]], ss1, add=True)
        o4 = pltpu.async_copy(ones16, cnt_sh.at[dc.at[0]], so1, add=True)
        s3.wait()
        o3.wait()
        pltpu.make_async_copy(src_hbm.at[wid, pl.ds(0, 2)], sa, semia).wait()
        pltpu.make_async_copy(dst_hbm.at[wid, pl.ds(0, 2)], da, semia).wait()
        pltpu.async_copy(x_hbm.at[sa.at[0]], rows.at[0], g0)
        pltpu.async_copy(src_hbm.at[wid, pl.ds(j + 8, 2)], sb, semib)
        pltpu.async_copy(dst_hbm.at[wid, pl.ds(j + 8, 2)], db, semib)
        pltpu.make_async_copy(x_hbm.at[sc_.at[1]], rows.at[2], g2).wait()
        s5 = pltpu.async_copy(rows.at[2], acc_sh.at[dc.at[1]], ss2, add=True)
        o5 = pltpu.async_copy(ones16, cnt_sh.at[dc.at[1]], so2, add=True)
        s4.wait()
        o4.wait()
        pltpu.async_copy(x_hbm.at[sa.at[1]], rows.at[1], g1)

        @pl.when(i < _NHEX - 1)
        def _():
            pltpu.async_copy(src_hbm.at[wid, pl.ds(j + 10, 2)], sc_, semic)
            pltpu.async_copy(dst_hbm.at[wid, pl.ds(j + 10, 2)], dc, semic)

        s5.wait()
        o5.wait()
        return carry

    lax.fori_loop(0, _NHEX, _hex, 0)
    jt = 6 * _NHEX
    pltpu.make_async_copy(x_hbm.at[sa.at[0]], rows.at[0], g0).wait()
    s0 = pltpu.async_copy(rows.at[0], acc_sh.at[da.at[0]], ss0, add=True)
    o0 = pltpu.async_copy(ones16, cnt_sh.at[da.at[0]], so0, add=True)
    pltpu.make_async_copy(src_hbm.at[wid, pl.ds(0, 2)], sb, semib).wait()
    pltpu.make_async_copy(dst_hbm.at[wid, pl.ds(0, 2)], db, semib).wait()
    pltpu.async_copy(x_hbm.at[sb.at[0]], rows.at[2], g2)
    pltpu.make_async_copy(x_hbm.at[sa.at[1]], rows.at[1], g1).wait()
    s1 = pltpu.async_copy(rows.at[1], acc_sh.at[da.at[1]], ss1, add=True)
    o1 = pltpu.async_copy(ones16, cnt_sh.at[da.at[1]], so1, add=True)
    s0.wait()
    o0.wait()
    pltpu.async_copy(x_hbm.at[sb.at[1]], rows.at[0], g0)
    pltpu.make_async_copy(x_hbm.at[sb.at[0]], rows.at[2], g2).wait()
    s2 = pltpu.async_copy(rows.at[2], acc_sh.at[db.at[0]], ss2, add=True)
    o2 = pltpu.async_copy(ones16, cnt_sh.at[db.at[0]], so2, add=True)
    s1.wait()
    o1.wait()
    pltpu.sync_copy(src_hbm.at[wid, pl.ds(_NCHK - 1, 1)], sa.at[pl.ds(0, 1)])
    pltpu.sync_copy(dst_hbm.at[wid, pl.ds(_NCHK - 1, 1)], da.at[pl.ds(0, 1)])
    pltpu.async_copy(x_hbm.at[sa.at[0]], rows.at[1], g1)
    pltpu.make_async_copy(x_hbm.at[sb.at[1]], rows.at[0], g0).wait()
    s3 = pltpu.async_copy(rows.at[0], acc_sh.at[db.at[1]], ss0, add=True)
    o3 = pltpu.async_copy(ones16, cnt_sh.at[db.at[1]], so0, add=True)
    s2.wait()
    o2.wait()
    pltpu.make_async_copy(x_hbm.at[sa.at[0]], rows.at[1], g1).wait()
    s4 = pltpu.async_copy(rows.at[1], acc_sh.at[da.at[0]], ss1, add=True)
    o4 = pltpu.async_copy(ones16, cnt_sh.at[da.at[0]], so1, add=True)
    s3.wait()
    o3.wait()
    s4.wait()
    o4.wait()
    plsc.subcore_barrier()

    pltpu.sync_copy(acc_sh.at[pl.ds(r0, _RPT)], p_out.at[c, pl.ds(r0, _RPT)])
    pltpu.sync_copy(cnt_sh.at[pl.ds(r0, _RPT)], cnt_out.at[c, pl.ds(r0, _RPT)])


@functools.partial(
    pl.kernel,
    mesh=_mesh,
    out_type=[
        jax.ShapeDtypeStruct((_NC, _NP, _F), jnp.float32),
        jax.ShapeDtypeStruct((_NC, _NP), jnp.float32),
    ],
    scratch_types=[
        pltpu.VMEM_SHARED((_NP, _F), jnp.float32),
        pltpu.VMEM_SHARED((_NP,), jnp.float32),
        pltpu.VMEM_SHARED((_NP,), jnp.float32),
        pltpu.VMEM((2, _CH), jnp.int32),
        pltpu.VMEM((2, _CH), jnp.int32),
        pltpu.VMEM((2, _CH), jnp.int32),
        pltpu.VMEM((2, _CH), jnp.int32),
        pltpu.VMEM((2, _CH), jnp.int32),
        pltpu.VMEM((2, _CH), jnp.int32),
        pltpu.VMEM((3, _CH, _F), jnp.float32),
        pltpu.VMEM((3, _CH), jnp.float32),
        pltpu.VMEM((_RPT,), jnp.float32),
        pltpu.VMEM((80, 16), jnp.float32),
        pltpu.VMEM((80, 16), jnp.float32),
        pltpu.VMEM((_NP,), jnp.float32),
        pltpu.SemaphoreType.DMA,
        pltpu.SemaphoreType.DMA,
        pltpu.SemaphoreType.DMA,
        pltpu.SemaphoreType.DMA,
        pltpu.SemaphoreType.DMA,
        pltpu.SemaphoreType.DMA,
        pltpu.SemaphoreType.DMA,
        pltpu.SemaphoreType.DMA,
        pltpu.SemaphoreType.DMA,
        pltpu.SemaphoreType.DMA,
        pltpu.SemaphoreType.DMA,
        pltpu.SemaphoreType.DMA,
    ],
    compiler_params=_scparams,
)
def _sc_segsum_cvec(src_hbm, dst_hbm, h_hbm, cnt_hbm, zrow_hbm,
                    q_out, cvec_out,
                    acc_sh, cvec_sh, inv_sh, sa, da, sb, db, sc_, dc, rows,
                    w, tmp1, cbuf0, cbuf1, invloc,
                    g0, g1, g2, ss0, ss1, ss2, sw0, sw1, sw2,
                    semia, semib, semic):
    c = lax.axis_index("c")
    s = lax.axis_index("s")
    wid = c * _NS + s
    r0 = s * _RPT

    pltpu.sync_copy(src_hbm.at[wid, pl.ds(0, 2)], sa)
    pltpu.sync_copy(dst_hbm.at[wid, pl.ds(0, 2)], da)
    pltpu.async_copy(src_hbm.at[wid, pl.ds(2, 2)], sb, semib)
    pltpu.async_copy(dst_hbm.at[wid, pl.ds(2, 2)], db, semib)
    pltpu.async_copy(src_hbm.at[wid, pl.ds(4, 2)], sc_, semic)
    pltpu.async_copy(dst_hbm.at[wid, pl.ds(4, 2)], dc, semic)
    pltpu.async_copy(h_hbm.at[sa.at[0]], rows.at[0], g0)
    pltpu.async_copy(h_hbm.at[sa.at[1]], rows.at[1], g1)

    zidx = jnp.zeros((16,), jnp.int32)
    i16 = lax.iota(jnp.int32, 16)
    for q in range(8):
        pltpu.sync_copy(cnt_hbm.at[0, pl.ds(r0 + q * 80, 80)], cbuf0)
        pltpu.sync_copy(cnt_hbm.at[1, pl.ds(r0 + q * 80, 80)], cbuf1)

        def _inv_grp(g, carry):
            ridx = i16 + g * 16
            c0 = plsc.load_gather(cbuf0, [ridx, zidx])
            c1 = plsc.load_gather(cbuf1, [ridx, zidx])
            tmp1[pl.ds(q * 80 + g * 16, 16)] = 1.0 / jnp.maximum(c0 + c1, 1.0)
            return carry

        lax.fori_loop(0, 5, _inv_grp, 0)
    pltpu.sync_copy(tmp1, inv_sh.at[pl.ds(r0, _RPT)])

    pltpu.sync_copy(zrow_hbm, acc_sh.at[pl.ds(r0, _RPT)])

    def _z1(i, carry):
        tmp1[pl.ds(i * 16, 16)] = jnp.zeros((16,), jnp.float32)
        return carry

    lax.fori_loop(0, _RPT // 16, _z1, 0)
    pltpu.sync_copy(tmp1, cvec_sh.at[pl.ds(r0, _RPT)])
    plsc.subcore_barrier()
    pltpu.sync_copy(inv_sh, invloc)

    def _wvals(b, dref, ch):
        for k in range(_CH // 16):
            d16 = dref[ch, pl.ds(k * 16, 16)]
            w[b, pl.ds(k * 16, 16)] = plsc.load_gather(invloc, [d16])

    def _chunk(b, sref, dref, ch, ssem, wsem):
        pltpu.make_async_copy(h_hbm.at[sref.at[ch]], rows.at[b], [g0, g1, g2][b]).wait()
        sD = pltpu.async_copy(rows.at[b], acc_sh.at[dref.at[ch]], ssem,
                              add=True)
        _wvals(b, dref, ch)
        wD = pltpu.async_copy(w.at[b], cvec_sh.at[sref.at[ch]], wsem,
                              add=True)
        return sD, wD

    def _hex(i, carry):
        j = 6 * i
        s0, w0 = _chunk(0, sa, da, 0, ss0, sw0)
        pltpu.make_async_copy(src_hbm.at[wid, pl.ds(0, 2)], sb, semib).wait()
        pltpu.make_async_copy(dst_hbm.at[wid, pl.ds(0, 2)], db, semib).wait()
        pltpu.async_copy(h_hbm.at[sb.at[0]], rows.at[2], g2)
        s1, w1 = _chunk(1, sa, da, 1, ss1, sw1)
        s0.wait()
        w0.wait()
        pltpu.async_copy(h_hbm.at[sb.at[1]], rows.at[0], g0)
        s2, w2 = _chunk(2, sb, db, 0, ss2, sw2)
        s1.wait()
        w1.wait()
        pltpu.make_async_copy(src_hbm.at[wid, pl.ds(0, 2)], sc_, semic).wait()
        pltpu.make_async_copy(dst_hbm.at[wid, pl.ds(0, 2)], dc, semic).wait()
        pltpu.async_copy(h_hbm.at[sc_.at[0]], rows.at[1], g1)
        pltpu.async_copy(src_hbm.at[wid, pl.ds(j + 6, 2)], sa, semia)
        pltpu.async_copy(dst_hbm.at[wid, pl.ds(j + 6, 2)], da, semia)
        s3, w3 = _chunk(0, sb, db, 1, ss0, sw0)
        s2.wait()
        w2.wait()
        pltpu.async_copy(h_hbm.at[sc_.at[1]], rows.at[2], g2)
        s4, w4 = _chunk(1, sc_, dc, 0, ss1, sw1)
        s3.wait()
        w3.wait()
        pltpu.make_async_copy(src_hbm.at[wid, pl.ds(0, 2)], sa, semia).wait()
        pltpu.make_async_copy(dst_hbm.at[wid, pl.ds(0, 2)], da, semia).wait()
        pltpu.async_copy(h_hbm.at[sa.at[0]], rows.at[0], g0)
        pltpu.async_copy(src_hbm.at[wid, pl.ds(j + 8, 2)], sb, semib)
        pltpu.async_copy(dst_hbm.at[wid, pl.ds(j + 8, 2)], db, semib)
        s5, w5 = _chunk(2, sc_, dc, 1, ss2, sw2)
        s4.wait()
        w4.wait()
        pltpu.async_copy(h_hbm.at[sa.at[1]], rows.at[1], g1)

        @pl.when(i < _NHEX - 1)
        def _():
            pltpu.async_copy(src_hbm.at[wid, pl.ds(j + 10, 2)], sc_, semic)
            pltpu.async_copy(dst_hbm.at[wid, pl.ds(j + 10, 2)], dc, semic)

        s5.wait()
        w5.wait()
        return carry

    lax.fori_loop(0, _NHEX, _hex, 0)
    s0, w0 = _chunk(0, sa, da, 0, ss0, sw0)
    pltpu.make_async_copy(src_hbm.at[wid, pl.ds(0, 2)], sb, semib).wait()
    pltpu.make_async_copy(dst_hbm.at[wid, pl.ds(0, 2)], db, semib).wait()
    pltpu.async_copy(h_hbm.at[sb.at[0]], rows.at[2], g2)
    s1, w1 = _chunk(1, sa, da, 1, ss1, sw1)
    s0.wait()
    w0.wait()
    pltpu.async_copy(h_hbm.at[sb.at[1]], rows.at[0], g0)
    s2, w2 = _chunk(2, sb, db, 0, ss2, sw2)
    s1.wait()
    w1.wait()
    pltpu.sync_copy(src_hbm.at[wid, pl.ds(_NCHK - 1, 1)], sa.at[pl.ds(0, 1)])
    pltpu.sync_copy(dst_hbm.at[wid, pl.ds(_NCHK - 1, 1)], da.at[pl.ds(0, 1)])
    pltpu.async_copy(h_hbm.at[sa.at[0]], rows.at[1], g1)
    s3, w3 = _chunk(0, sb, db, 1, ss0, sw0)
    s2.wait()
    w2.wait()
    s4, w4 = _chunk(1, sa, da, 0, ss1, sw1)
    s3.wait()
    w3.wait()
    s4.wait()
    w4.wait()
    plsc.subcore_barrier()

    pltpu.sync_copy(acc_sh.at[pl.ds(r0, _RPT)], q_out.at[c, pl.ds(r0, _RPT)])
    pltpu.sync_copy(cvec_sh.at[pl.ds(r0, _RPT)], cvec_out.at[c, pl.ds(r0, _RPT)])


def _dotT(a, b):
    return lax.dot_general(a, b, (((1,), (1,)), ((), ())),
                           preferred_element_type=jnp.float32)


def _dense1_body(p_ref, cnt_ref, x_ref, wl_ref, bl_ref, wr_ref, h_ref):
    cnt = cnt_ref[0][:, 0:1] + cnt_ref[1][:, 0:1]
    inv = 1.0 / jnp.maximum(cnt, 1.0)
    agg = (p_ref[0] + p_ref[1]) * inv
    h = _dotT(agg, wl_ref[...]) + _dotT(x_ref[...], wr_ref[...]) + bl_ref[...]
    h_ref[...] = jnp.maximum(h, 0.0)


def _dense2_body(q_ref, cv_ref, h1_ref, cnt_ref, wl2_ref, bl2_ref, wr2_ref,
                 wl3_ref, bl3_ref, wr3_ref, wc_ref, bc_ref,
                 out_ref, g1_acc, g2_acc):
    i = pl.program_id(0)
    cnt = cnt_ref[0][:, 0:1] + cnt_ref[1][:, 0:1]
    inv = 1.0 / jnp.maximum(cnt, 1.0)
    agg = (q_ref[0] + q_ref[1]) * inv
    h2 = _dotT(agg, wl2_ref[...]) + _dotT(h1_ref[...], wr2_ref[...]) + bl2_ref[...]
    h2 = jnp.maximum(h2, 0.0)
    cv = cv_ref[:, 0:1] + cv_ref[:, 1:2]
    part1 = jnp.sum(cv * h2, axis=0, keepdims=True)
    part2 = jnp.sum(h2, axis=0, keepdims=True)

    @pl.when(i == 0)
    def _():
        g1_acc[...] = jnp.zeros_like(g1_acc)
        g2_acc[...] = jnp.zeros_like(g2_acc)

    g1_acc[...] += part1
    g2_acc[...] += part2

    @pl.when(i == _G - 1)
    def _():
        g1 = g1_acc[...] * (1.0 / _N)
        g2 = g2_acc[...] * (1.0 / _N)
        gm = _dotT(g1, wl3_ref[...]) + bl3_ref[...] + _dotT(g2, wr3_ref[...])
        out_ref[...] = _dotT(gm, wc_ref[...]) + bc_ref[...]


def kernel(x, edge_index, Wl1, bl1, Wr1, Wl2, bl2, Wr2, Wl3, bl3, Wr3, Wc, bc):
    src_r = edge_index[0].reshape(_NW, _NCHK, _CH)
    dst_r = edge_index[1].reshape(_NW, _NCHK, _CH)
    zrow = jnp.zeros((_RPT, _F), jnp.float32)
    zc16 = jnp.zeros((_RPT, 16), jnp.float32)

    p1, cnt16 = _sc_segsum_counts(src_r, dst_r, x, zrow, zc16)

    wfull = pl.BlockSpec((_F, _F), lambda i: (0, 0))
    bfull = pl.BlockSpec((1, _F), lambda i: (0, 0))
    rowblk = pl.BlockSpec((_R, _F), lambda i: (i, 0))
    pblk = pl.BlockSpec((_NC, _R, _F), lambda i: (0, i, 0))
    cblk = pl.BlockSpec((_NC, _R, 16), lambda i: (0, i, 0))

    h1 = pl.pallas_call(
        _dense1_body,
        grid=(_G,),
        in_specs=[pblk, cblk, rowblk, wfull, bfull, wfull],
        out_specs=rowblk,
        out_shape=jax.ShapeDtypeStruct((_N, _F), jnp.float32),
    )(p1, cnt16, x, Wl1, bl1.reshape(1, _F), Wr1)

    q2, cvec = _sc_segsum_cvec(src_r, dst_r, h1, cnt16, zrow)

    cvblk = pl.BlockSpec((_R, _NC), lambda i: (i, 0))
    out = pl.pallas_call(
        _dense2_body,
        grid=(_G,),
        in_specs=[pblk, cvblk, rowblk, cblk, wfull, bfull, wfull,
                  wfull, bfull, wfull,
                  pl.BlockSpec((Wc.shape[0], _F), lambda i: (0, 0)),
                  pl.BlockSpec((1, Wc.shape[0]), lambda i: (0, 0))],
        out_specs=pl.BlockSpec((1, Wc.shape[0]), lambda i: (0, 0)),
        out_shape=jax.ShapeDtypeStruct((1, Wc.shape[0]), jnp.float32),
        scratch_shapes=[pltpu.VMEM((1, _F), jnp.float32),
                        pltpu.VMEM((1, _F), jnp.float32)],
    )(q2, cvec.T, h1, cnt16, Wl2, bl2.reshape(1, _F), Wr2,
      Wl3, bl3.reshape(1, _F), Wr3, Wc, bc.reshape(1, -1))

    return out
